# baseline (speedup 1.0000x reference)
import dataclasses
import functools

import jax
import jax.numpy as jnp
from jax import lax
from jax.experimental import pallas as pl
from jax.experimental.pallas import tpu as pltpu
from jax.experimental.pallas import tpu_sc as plsc

N = 10000
D = 128
NC, NS = 2, 16
NW = NC * NS
C = 128
NQ = 5
QC = 16
CHUNKS = NQ * QC
EPW = C * CHUNKS
E_PAD = NW * EPW

ROWS_PW = 624

_sc_mesh = plsc.VectorSubcoreMesh(core_axis_name="c", subcore_axis_name="s")

_sc_params = pltpu.CompilerParams()
if "needs_layout_passes" in pltpu.CompilerParams.__dataclass_fields__:
    _sc_params = dataclasses.replace(_sc_params, needs_layout_passes=False)


@functools.partial(
    pl.kernel,
    out_type=jax.ShapeDtypeStruct((NC, N, D), jnp.float32),
    mesh=_sc_mesh,
    compiler_params=_sc_params,
    scratch_types=[
        pltpu.VMEM((QC, C), jnp.int32),
        pltpu.VMEM((QC, C), jnp.int32),
        pltpu.VMEM((QC, C), jnp.int32),
        pltpu.VMEM((QC, C), jnp.int32),
        pltpu.VMEM((QC, C), jnp.float32),
        pltpu.VMEM((QC, C), jnp.float32),
        pltpu.VMEM((C, D), jnp.float32),
        pltpu.VMEM((C, D), jnp.float32),
        pltpu.VMEM_SHARED((N, D), jnp.float32),
        pltpu.SemaphoreType.DMA,
        pltpu.SemaphoreType.DMA,
        pltpu.SemaphoreType.DMA,
        pltpu.SemaphoreType.DMA,
        pltpu.SemaphoreType.DMA,
    ],
)
def _spmm(table_hbm, src_hbm, dst_hbm, vals_hbm, out_hbm,
          src_v0, src_v1, dst_v0, dst_v1, vals_v0, vals_v1, rows0, rows1, acc,
          sem0, sem1, sem_idx, ssem0, ssem1):
    c = lax.axis_index("c")
    s = lax.axis_index("s")
    wid = c * NS + s

    src_b = (src_v0, src_v1)
    dst_b = (dst_v0, dst_v1)
    vals_b = (vals_v0, vals_v1)

    def _stage_slab(q, par):
        pltpu.async_copy(src_hbm.at[wid, q], src_b[par], sem_idx)
        pltpu.async_copy(dst_hbm.at[wid, q], dst_b[par], sem_idx)
        pltpu.async_copy(vals_hbm.at[wid, q], vals_b[par], sem_idx)

    def _wait_slab():
        pltpu.make_async_copy(src_hbm.at[wid, 0], src_v0, sem_idx).wait()
        pltpu.make_async_copy(dst_hbm.at[wid, 0], dst_v0, sem_idx).wait()
        pltpu.make_async_copy(vals_hbm.at[wid, 0], vals_v0, sem_idx).wait()

    _stage_slab(0, 0)

    zero16 = jnp.zeros((16,), jnp.float32)

    @pl.loop(0, C)
    def _(r):
        for g in range(8):
            rows0[r, pl.ds(g * 16, 16)] = zero16

    base = s * ROWS_PW
    for i in range(4):
        pltpu.sync_copy(rows0, acc.at[pl.ds(base + i * C, C)])
    pltpu.sync_copy(rows0.at[pl.ds(0, ROWS_PW - 4 * C)],
                    acc.at[pl.ds(base + 4 * C, ROWS_PW - 4 * C)])

    @pl.when(s == 0)
    def _():
        pltpu.sync_copy(rows0.at[pl.ds(0, N - NS * ROWS_PW)],
                        acc.at[pl.ds(NS * ROWS_PW, N - NS * ROWS_PW)])

    _wait_slab()
    plsc.subcore_barrier()

    def _wait_rows(buf, sem):
        pltpu.make_async_copy(table_hbm.at[pl.ds(0, C)], buf, sem).wait()

    def _wait_scatter(buf, ssem):
        pltpu.make_async_copy(buf, acc.at[dst_v0.at[0]], ssem).wait()

    _dnums = lax.GatherDimensionNumbers(
        offset_dims=(), collapsed_slice_dims=(0,), start_index_map=(0,))

    def _scale_half(par, k, buf, h):
        @pl.loop(h * (C // 2), (h + 1) * (C // 2), step=16)
        def _(e0):
            v16 = vals_b[par][k, pl.ds(e0, 16)]
            for i in range(16):
                idx = jnp.full((16, 1), i, jnp.int32)
                vsp = lax.gather(v16, idx, _dnums, (1,),
                                 mode=lax.GatherScatterMode.PROMISE_IN_BOUNDS)
                for g in range(8):
                    sl = pl.ds(g * 16, 16)
                    buf[e0 + i, sl] = buf[e0 + i, sl] * vsp

    for q in range(NQ):
        par = q % 2
        if q + 1 < NQ:
            _stage_slab(q + 1, 1 - par)
        pltpu.async_copy(table_hbm.at[src_b[par].at[0]], rows0, sem0)

        @pl.loop(0, QC, step=2)
        def _(k):
            _wait_rows(rows0, sem0)
            _scale_half(par, k, rows0, 0)

            if q == 0:
                @pl.when(k > 0)
                def _():
                    _wait_scatter(rows1, ssem1)
            else:
                _wait_scatter(rows1, ssem1)
            pltpu.async_copy(table_hbm.at[src_b[par].at[k + 1]], rows1, sem1)

            _scale_half(par, k, rows0, 1)
            pltpu.async_copy(rows0, acc.at[dst_b[par].at[k]], ssem0,
                             add=True)

            _wait_rows(rows1, sem1)
            _scale_half(par, k + 1, rows1, 0)
            _wait_scatter(rows0, ssem0)

            @pl.when(k + 2 < QC)
            def _():
                pltpu.async_copy(table_hbm.at[src_b[par].at[k + 2]], rows0,
                                 sem0)

            _scale_half(par, k + 1, rows1, 1)
            pltpu.async_copy(rows1, acc.at[dst_b[par].at[k + 1]], ssem1,
                             add=True)

        if q + 1 < NQ:
            _wait_slab()

    _wait_scatter(rows1, ssem1)
    plsc.subcore_barrier()
    pltpu.sync_copy(acc.at[pl.ds(base, ROWS_PW)],
                    out_hbm.at[c].at[pl.ds(base, ROWS_PW)])

    @pl.when(s == 0)
    def _():
        pltpu.sync_copy(acc.at[pl.ds(NS * ROWS_PW, N - NS * ROWS_PW)],
                        out_hbm.at[c].at[pl.ds(NS * ROWS_PW, N - NS * ROWS_PW)])


_RB = 1000


def _combine_body(hp_ref, o_ref):
    o_ref[...] = hp_ref[0] + hp_ref[1]


_combine = pl.pallas_call(
    _combine_body,
    grid=(N // _RB,),
    in_specs=[pl.BlockSpec((NC, _RB, D), lambda i: (0, i, 0))],
    out_specs=pl.BlockSpec((_RB, D), lambda i: (i, 0)),
    out_shape=jax.ShapeDtypeStruct((N, D), jnp.float32),
)


def _dense_body(hp_ref, x_ref, w_ref, b_ref, o_ref):
    h = hp_ref[0] + hp_ref[1]
    w = w_ref[...]
    dn = (((1,), (1,)), ((), ()))
    acc = lax.dot_general(h, w[:, :D], dn, preferred_element_type=jnp.float32)
    acc = acc + lax.dot_general(x_ref[...], w[:, D:], dn,
                                preferred_element_type=jnp.float32)
    o_ref[...] = jnp.maximum(acc + b_ref[...], 0.0)


_dense = pl.pallas_call(
    _dense_body,
    grid=(N // _RB,),
    in_specs=[
        pl.BlockSpec((NC, _RB, D), lambda i: (0, i, 0)),
        pl.BlockSpec((_RB, D), lambda i: (i, 0)),
        pl.BlockSpec((D, 2 * D), lambda i: (0, 0)),
        pl.BlockSpec((1, D), lambda i: (0, 0)),
    ],
    out_specs=pl.BlockSpec((_RB, D), lambda i: (i, 0)),
    out_shape=jax.ShapeDtypeStruct((N, D), jnp.float32),
)


def kernel(ufea, vfea, uv_edges, uv_vals, Wu, bu, Wi, bi):
    row = uv_edges[0].astype(jnp.int32)
    col = uv_edges[1].astype(jnp.int32)
    vals = uv_vals.astype(jnp.float32)
    pad = E_PAD - row.shape[0]
    rowp = jnp.concatenate([row, jnp.zeros((pad,), jnp.int32)])
    colp = jnp.concatenate([col, jnp.zeros((pad,), jnp.int32)])
    valsp = jnp.concatenate([vals, jnp.zeros((pad,), jnp.float32)])
    rowp = rowp.reshape(NW, NQ, QC, C)
    colp = colp.reshape(NW, NQ, QC, C)
    valsp = valsp.reshape(NW, NQ, QC, C)

    U1p = _spmm(ufea, rowp, colp, valsp)
    V1p = _spmm(vfea, colp, rowp, valsp)
    U1 = _combine(U1p)
    V1 = _combine(V1p)
    U2p = _spmm(U1, colp, rowp, valsp)
    V2p = _spmm(V1, rowp, colp, valsp)

    user = _dense(U2p, ufea, Wu, bu.reshape(1, D))
    item = _dense(V2p, vfea, Wi, bi.reshape(1, D))
    return (user, item)


# speedup vs baseline: 1.5467x; 1.5467x over previous
import dataclasses
import functools

import jax
import jax.numpy as jnp
from jax import lax
from jax.experimental import pallas as pl
from jax.experimental.pallas import tpu as pltpu
from jax.experimental.pallas import tpu_sc as plsc

N = 10000
D = 128
NC, NS = 2, 16
C = 128
NQ = 10
QC = 16
CHUNKS = NQ * QC
EPW = C * CHUNKS
E_PAD = NS * EPW

ROWS_PW = 624

_sc_mesh = plsc.VectorSubcoreMesh(core_axis_name="c", subcore_axis_name="s")

_sc_params = pltpu.CompilerParams()
if "needs_layout_passes" in pltpu.CompilerParams.__dataclass_fields__:
    _sc_params = dataclasses.replace(_sc_params, needs_layout_passes=False)


@functools.partial(
    pl.kernel,
    out_type=[jax.ShapeDtypeStruct((NC, N, D), jnp.float32),
              jax.ShapeDtypeStruct((NC, N, D), jnp.float32)],
    mesh=_sc_mesh,
    compiler_params=_sc_params,
    scratch_types=[
        pltpu.VMEM((QC, C), jnp.int32),
        pltpu.VMEM((QC, C), jnp.int32),
        pltpu.VMEM((QC, C), jnp.int32),
        pltpu.VMEM((QC, C), jnp.int32),
        pltpu.VMEM((QC, C), jnp.float32),
        pltpu.VMEM((QC, C), jnp.float32),
        pltpu.VMEM((C, D), jnp.float32),
        pltpu.VMEM((C, D), jnp.float32),
        pltpu.VMEM_SHARED((N, D), jnp.float32),
        pltpu.SemaphoreType.DMA,
        pltpu.SemaphoreType.DMA,
        pltpu.SemaphoreType.DMA,
    ],
)
def _dlgcn(tab_hbm, idx_hbm, vals_hbm, stage_hbm, final_hbm,
           src_v0, src_v1, dst_v0, dst_v1, vals_v0, vals_v1, rows0, rows1,
           acc, sem0, sem1, sem_idx):
    c = lax.axis_index("c")
    s = lax.axis_index("s")

    src_b = (src_v0, src_v1)
    dst_b = (dst_v0, dst_v1)
    vals_b = (vals_v0, vals_v1)

    base = s * ROWS_PW
    zero16 = jnp.zeros((16,), jnp.float32)

    def _run_phase(table, srcsel, dstsel, out):
        def _stage_slab(q, par):
            pltpu.async_copy(idx_hbm.at[srcsel, s, q], src_b[par], sem_idx)
            pltpu.async_copy(idx_hbm.at[dstsel, s, q], dst_b[par], sem_idx)
            pltpu.async_copy(vals_hbm.at[s, q], vals_b[par], sem_idx)

        def _wait_slab():
            pltpu.make_async_copy(idx_hbm.at[0, 0, 0], src_v0, sem_idx).wait()
            pltpu.make_async_copy(idx_hbm.at[0, 0, 0], dst_v0, sem_idx).wait()
            pltpu.make_async_copy(vals_hbm.at[0, 0], vals_v0, sem_idx).wait()

        _stage_slab(0, 0)

        @pl.loop(0, C)
        def _(r):
            for g in range(8):
                rows0[r, pl.ds(g * 16, 16)] = zero16

        for i in range(4):
            pltpu.sync_copy(rows0, acc.at[pl.ds(base + i * C, C)])
        pltpu.sync_copy(rows0.at[pl.ds(0, ROWS_PW - 4 * C)],
                        acc.at[pl.ds(base + 4 * C, ROWS_PW - 4 * C)])

        @pl.when(s == 0)
        def _():
            pltpu.sync_copy(rows0.at[pl.ds(0, N - NS * ROWS_PW)],
                            acc.at[pl.ds(NS * ROWS_PW, N - NS * ROWS_PW)])

        _wait_slab()
        plsc.subcore_barrier()

        def _wait_rows(buf, sem):
            pltpu.make_async_copy(table.at[pl.ds(0, C)], buf, sem).wait()

        def _process(par, k, buf):
            kvec = jnp.full((16,), k, jnp.int32)

            @pl.loop(0, C)
            def _(e):
                evec = jnp.full((16,), e, jnp.int32)
                vsp = plsc.load_gather(vals_b[par], [kvec, evec])
                for g in range(8):
                    sl = pl.ds(g * 16, 16)
                    buf[e, sl] = buf[e, sl] * vsp

            pltpu.sync_copy(buf, acc.at[dst_b[par].at[k]], add=True)

        for q in range(NQ):
            par = q % 2
            if q + 1 < NQ:
                _stage_slab(q + 1, 1 - par)
            pltpu.async_copy(table.at[src_b[par].at[0]], rows0, sem0)

            @pl.loop(0, QC, step=2)
            def _(k):
                pltpu.async_copy(table.at[src_b[par].at[k + 1]], rows1, sem1)
                _wait_rows(rows0, sem0)
                _process(par, k, rows0)

                @pl.when(k + 2 < QC)
                def _():
                    pltpu.async_copy(table.at[src_b[par].at[k + 2]], rows0,
                                     sem0)

                _wait_rows(rows1, sem1)
                _process(par, k + 1, rows1)

            if q + 1 < NQ:
                _wait_slab()

        plsc.subcore_barrier()
        pltpu.sync_copy(acc.at[pl.ds(base, ROWS_PW)],
                        out.at[pl.ds(base, ROWS_PW)])

        @pl.when(s == 0)
        def _():
            pltpu.sync_copy(acc.at[pl.ds(NS * ROWS_PW, N - NS * ROWS_PW)],
                            out.at[pl.ds(NS * ROWS_PW, N - NS * ROWS_PW)])

        plsc.subcore_barrier()

    _run_phase(tab_hbm.at[c], c, 1 - c, stage_hbm.at[c])
    _run_phase(stage_hbm.at[c], 1 - c, c, final_hbm.at[c])


_RB = 1000


def _dense_body(h_ref, x_ref, w_ref, b_ref, o_ref):
    h = h_ref[0]
    x = x_ref[0]
    w = w_ref[0]
    dn = (((1,), (1,)), ((), ()))
    acc = lax.dot_general(h, w[:, :D], dn, preferred_element_type=jnp.float32)
    acc = acc + lax.dot_general(x, w[:, D:], dn,
                                preferred_element_type=jnp.float32)
    o_ref[0] = jnp.maximum(acc + b_ref[0], 0.0)


_dense = pl.pallas_call(
    _dense_body,
    grid=(2, N // _RB),
    in_specs=[
        pl.BlockSpec((1, _RB, D), lambda j, i: (j, i, 0)),
        pl.BlockSpec((1, _RB, D), lambda j, i: (j, i, 0)),
        pl.BlockSpec((1, D, 2 * D), lambda j, i: (j, 0, 0)),
        pl.BlockSpec((1, 1, D), lambda j, i: (j, 0, 0)),
    ],
    out_specs=pl.BlockSpec((1, _RB, D), lambda j, i: (j, i, 0)),
    out_shape=jax.ShapeDtypeStruct((2, N, D), jnp.float32),
)


def kernel(ufea, vfea, uv_edges, uv_vals, Wu, bu, Wi, bi):
    row = uv_edges[0].astype(jnp.int32)
    col = uv_edges[1].astype(jnp.int32)
    vals = uv_vals.astype(jnp.float32)
    pad = E_PAD - row.shape[0]
    rowp = jnp.concatenate([row, jnp.zeros((pad,), jnp.int32)])
    colp = jnp.concatenate([col, jnp.zeros((pad,), jnp.int32)])
    valsp = jnp.concatenate([vals, jnp.zeros((pad,), jnp.float32)])
    idx = jnp.stack([rowp, colp]).reshape(2, NS, NQ, QC, C)
    valsp = valsp.reshape(NS, NQ, QC, C)

    tab = jnp.stack([ufea, vfea])
    _, final = _dlgcn(tab, idx, valsp)

    w = jnp.stack([Wu, Wi])
    b = jnp.stack([bu, bi]).reshape(2, 1, D)
    out = _dense(final, tab, w, b)
    return (out[0], out[1])


# speedup vs baseline: 1.5753x; 1.0185x over previous
import dataclasses
import functools

import jax
import jax.numpy as jnp
from jax import lax
from jax.experimental import pallas as pl
from jax.experimental.pallas import tpu as pltpu
from jax.experimental.pallas import tpu_sc as plsc

N = 10000
D = 128
NC, NS = 2, 16
C = 128
NQ = 10
QC = 16
CHUNKS = NQ * QC
EPW = C * CHUNKS
E_PAD = NS * EPW

ROWS_PW = 624

_dnums = lax.GatherDimensionNumbers(
    offset_dims=(), collapsed_slice_dims=(0,), start_index_map=(0,))

_sc_mesh = plsc.VectorSubcoreMesh(core_axis_name="c", subcore_axis_name="s")

_sc_params = pltpu.CompilerParams()
if "needs_layout_passes" in pltpu.CompilerParams.__dataclass_fields__:
    _sc_params = dataclasses.replace(_sc_params, needs_layout_passes=False)


@functools.partial(
    pl.kernel,
    out_type=[jax.ShapeDtypeStruct((NC, N, D), jnp.float32),
              jax.ShapeDtypeStruct((NC, N, D), jnp.float32)],
    mesh=_sc_mesh,
    compiler_params=_sc_params,
    scratch_types=[
        pltpu.VMEM((QC, C), jnp.int32),
        pltpu.VMEM((QC, C), jnp.int32),
        pltpu.VMEM((QC, C), jnp.int32),
        pltpu.VMEM((QC, C), jnp.int32),
        pltpu.VMEM((QC, C), jnp.float32),
        pltpu.VMEM((QC, C), jnp.float32),
        pltpu.VMEM((C, D), jnp.float32),
        pltpu.VMEM((C, D), jnp.float32),
        pltpu.VMEM_SHARED((N, D), jnp.float32),
        pltpu.SemaphoreType.DMA,
        pltpu.SemaphoreType.DMA,
        pltpu.SemaphoreType.DMA,
        pltpu.SemaphoreType.DMA,
        pltpu.SemaphoreType.DMA,
    ],
)
def _dlgcn(tab_hbm, idx_hbm, vals_hbm, stage_hbm, final_hbm,
           src_v0, src_v1, dst_v0, dst_v1, vals_v0, vals_v1, rows0, rows1,
           acc, sem0, sem1, sem_idx, ssem0, ssem1):
    c = lax.axis_index("c")
    s = lax.axis_index("s")

    src_b = (src_v0, src_v1)
    dst_b = (dst_v0, dst_v1)
    vals_b = (vals_v0, vals_v1)

    base = s * ROWS_PW
    zero16 = jnp.zeros((16,), jnp.float32)

    def _run_phase(table, srcsel, dstsel, out):
        def _stage_slab(q, par):
            pltpu.async_copy(idx_hbm.at[srcsel, s, q], src_b[par], sem_idx)
            pltpu.async_copy(idx_hbm.at[dstsel, s, q], dst_b[par], sem_idx)
            pltpu.async_copy(vals_hbm.at[s, q], vals_b[par], sem_idx)

        def _wait_slab():
            pltpu.make_async_copy(idx_hbm.at[0, 0, 0], src_v0, sem_idx).wait()
            pltpu.make_async_copy(idx_hbm.at[0, 0, 0], dst_v0, sem_idx).wait()
            pltpu.make_async_copy(vals_hbm.at[0, 0], vals_v0, sem_idx).wait()

        _stage_slab(0, 0)

        @pl.loop(0, C)
        def _(r):
            for g in range(8):
                rows0[r, pl.ds(g * 16, 16)] = zero16

        for i in range(4):
            pltpu.sync_copy(rows0, acc.at[pl.ds(base + i * C, C)])
        pltpu.sync_copy(rows0.at[pl.ds(0, ROWS_PW - 4 * C)],
                        acc.at[pl.ds(base + 4 * C, ROWS_PW - 4 * C)])

        @pl.when(s == 0)
        def _():
            pltpu.sync_copy(rows0.at[pl.ds(0, N - NS * ROWS_PW)],
                            acc.at[pl.ds(NS * ROWS_PW, N - NS * ROWS_PW)])

        _wait_slab()
        plsc.subcore_barrier()

        def _wait_rows(buf, sem):
            pltpu.make_async_copy(table.at[pl.ds(0, C)], buf, sem).wait()

        def _wait_scatter(buf, ssem):
            pltpu.make_async_copy(buf, acc.at[dst_v0.at[0]], ssem).wait()

        def _scale_half(par, k, buf, h):
            @pl.loop(h * (C // 2), (h + 1) * (C // 2), step=16)
            def _(e0):
                v16 = vals_b[par][k, pl.ds(e0, 16)]
                for i in range(16):
                    idx = jnp.full((16, 1), i, jnp.int32)
                    vsp = lax.gather(
                        v16, idx, _dnums, (1,),
                        mode=lax.GatherScatterMode.PROMISE_IN_BOUNDS)
                    for g in range(8):
                        sl = pl.ds(g * 16, 16)
                        buf[e0 + i, sl] = buf[e0 + i, sl] * vsp

        def _slab_chunks(q, par, guard_first):
            @pl.loop(0, QC, step=2)
            def _(k):
                _wait_rows(rows0, sem0)
                _scale_half(par, k, rows0, 0)

                if guard_first:
                    @pl.when((q > 0) | (k > 0))
                    def _():
                        _wait_scatter(rows1, ssem1)
                else:
                    _wait_scatter(rows1, ssem1)
                pltpu.async_copy(table.at[src_b[par].at[k + 1]], rows1, sem1)

                _scale_half(par, k, rows0, 1)
                pltpu.async_copy(rows0, acc.at[dst_b[par].at[k]], ssem0,
                                 add=True)

                _wait_rows(rows1, sem1)
                _scale_half(par, k + 1, rows1, 0)
                _wait_scatter(rows0, ssem0)

                @pl.when(k + 2 < QC)
                def _():
                    pltpu.async_copy(table.at[src_b[par].at[k + 2]], rows0,
                                     sem0)

                _scale_half(par, k + 1, rows1, 1)
                pltpu.async_copy(rows1, acc.at[dst_b[par].at[k + 1]], ssem1,
                                 add=True)

        pltpu.async_copy(table.at[src_v0.at[0]], rows0, sem0)

        @pl.loop(0, NQ, step=2)
        def _(q):
            _stage_slab(q + 1, 1)
            _slab_chunks(q, 0, True)
            _wait_slab()
            pltpu.async_copy(table.at[src_v1.at[0]], rows0, sem0)

            @pl.when(q + 2 < NQ)
            def _():
                _stage_slab(q + 2, 0)

            _slab_chunks(q, 1, False)

            @pl.when(q + 2 < NQ)
            def _():
                _wait_slab()
                pltpu.async_copy(table.at[src_v0.at[0]], rows0, sem0)

        _wait_scatter(rows1, ssem1)
        plsc.subcore_barrier()
        pltpu.sync_copy(acc.at[pl.ds(base, ROWS_PW)],
                        out.at[pl.ds(base, ROWS_PW)])

        @pl.when(s == 0)
        def _():
            pltpu.sync_copy(acc.at[pl.ds(NS * ROWS_PW, N - NS * ROWS_PW)],
                            out.at[pl.ds(NS * ROWS_PW, N - NS * ROWS_PW)])

        plsc.subcore_barrier()

    _run_phase(tab_hbm.at[c], c, 1 - c, stage_hbm.at[c])
    _run_phase(stage_hbm.at[c], 1 - c, c, final_hbm.at[c])


_RB = 1000


def _dense_body(h_ref, x_ref, w_ref, b_ref, o_ref):
    h = h_ref[0]
    x = x_ref[0]
    w = w_ref[0]
    dn = (((1,), (1,)), ((), ()))
    acc = lax.dot_general(h, w[:, :D], dn, preferred_element_type=jnp.float32)
    acc = acc + lax.dot_general(x, w[:, D:], dn,
                                preferred_element_type=jnp.float32)
    o_ref[0] = jnp.maximum(acc + b_ref[0], 0.0)


_dense = pl.pallas_call(
    _dense_body,
    grid=(2, N // _RB),
    in_specs=[
        pl.BlockSpec((1, _RB, D), lambda j, i: (j, i, 0)),
        pl.BlockSpec((1, _RB, D), lambda j, i: (j, i, 0)),
        pl.BlockSpec((1, D, 2 * D), lambda j, i: (j, 0, 0)),
        pl.BlockSpec((1, 1, D), lambda j, i: (j, 0, 0)),
    ],
    out_specs=pl.BlockSpec((1, _RB, D), lambda j, i: (j, i, 0)),
    out_shape=jax.ShapeDtypeStruct((2, N, D), jnp.float32),
)


def kernel(ufea, vfea, uv_edges, uv_vals, Wu, bu, Wi, bi):
    row = uv_edges[0].astype(jnp.int32)
    col = uv_edges[1].astype(jnp.int32)
    vals = uv_vals.astype(jnp.float32)
    pad = E_PAD - row.shape[0]
    rowp = jnp.concatenate([row, jnp.zeros((pad,), jnp.int32)])
    colp = jnp.concatenate([col, jnp.zeros((pad,), jnp.int32)])
    valsp = jnp.concatenate([vals, jnp.zeros((pad,), jnp.float32)])
    idx = jnp.stack([rowp, colp]).reshape(2, NS, NQ, QC, C)
    valsp = valsp.reshape(NS, NQ, QC, C)

    tab = jnp.stack([ufea, vfea])
    _, final = _dlgcn(tab, idx, valsp)

    w = jnp.stack([Wu, Wi])
    b = jnp.stack([bu, bi]).reshape(2, 1, D)
    out = _dense(final, tab, w, b)
    return (out[0], out[1])
